# Initial kernel scaffold; baseline (speedup 1.0000x reference)
import jax
import jax.numpy as jnp
from jax.experimental import pallas as pl


def kernel(inputs, edge_index, W0, al0, ar0, W1, al1, ar1, W2, al2, ar2):
    raise NotImplementedError("write your pallas kernel here")


# speedup vs baseline: 1.0071x; 1.0071x over previous
import functools

import jax
import jax.numpy as jnp
from jax.experimental import pallas as pl

N = 10000
E = 320000
D_HID = 64
HEADS = 8
N_CLASSES = 40
NEG_SLOPE = 0.2

_BN = 1000


def _proj_body(x_ref, w_ref, o_ref):
    o_ref[...] = jnp.dot(x_ref[...], w_ref[...],
                         preferred_element_type=jnp.float32)


def _project(x, w):
    d_in, d_out = w.shape
    return pl.pallas_call(
        _proj_body,
        grid=(N // _BN,),
        in_specs=[
            pl.BlockSpec((_BN, d_in), lambda i: (i, 0)),
            pl.BlockSpec((d_in, d_out), lambda i: (0, 0)),
        ],
        out_specs=pl.BlockSpec((_BN, d_out), lambda i: (i, 0)),
        out_shape=jax.ShapeDtypeStruct((N, d_out), jnp.float32),
    )(x, w)


def _gat_layer(h, src, dst, W, al, ar, n_heads, f_out, apply_act):
    feat = _project(h, W).reshape(N, n_heads, f_out)
    el = jnp.sum(feat * al[None, :, :], axis=-1)
    er = jnp.sum(feat * ar[None, :, :], axis=-1)
    e = jax.nn.leaky_relu(el[src] + er[dst], NEG_SLOPE)
    m = jax.ops.segment_max(e, dst, num_segments=N)
    m = jnp.where(jnp.isfinite(m), m, 0.0)
    a = jnp.exp(e - m[dst])
    denom = jax.ops.segment_sum(a, dst, num_segments=N)
    msg = jax.ops.segment_sum(a[:, :, None] * feat[src], dst, num_segments=N)
    out = msg / (denom[:, :, None] + 1e-9)
    if apply_act:
        out = jax.nn.elu(out)
    return out


def kernel(inputs, edge_index, W0, al0, ar0, W1, al1, ar1, W2, al2, ar2):
    src = edge_index[0]
    dst = edge_index[1]
    h = _gat_layer(inputs, src, dst, W0, al0, ar0, HEADS, D_HID, True)
    h = h.reshape(N, HEADS * D_HID)
    h = _gat_layer(h, src, dst, W1, al1, ar1, HEADS, D_HID, True)
    h = h.reshape(N, HEADS * D_HID)
    logits = _gat_layer(h, src, dst, W2, al2, ar2, 1, N_CLASSES, False)
    logits = logits.mean(axis=1)
    return jax.nn.log_softmax(logits, axis=1)
